# baseline (speedup 1.0000x reference)
import functools

import jax
import jax.numpy as jnp
from jax.experimental import pallas as pl
from jax.experimental.pallas import tpu as pltpu

E = 64
K = 2
D = 1024
F = 768
T = 128
A = T * K
R = 8
NT = (A - E) // R + E
PR = NT * R


def _router_kernel(x_ref, rw_ref, px_ref, ct_ref, te_ref):
    x = x_ref[...]
    logits = jnp.dot(x, rw_ref[...], preferred_element_type=jnp.float32)
    probs = jax.nn.softmax(logits, axis=-1)

    col = jax.lax.broadcasted_iota(jnp.int32, (T, E), 1)
    i1 = jnp.argmax(probs, axis=1).reshape(T, 1)
    oh1 = (col == i1)
    m1 = jnp.sum(jnp.where(oh1, probs, 0.0), axis=1).reshape(T, 1)
    probs2 = jnp.where(oh1, -1.0, probs)
    i2 = jnp.argmax(probs2, axis=1).reshape(T, 1)
    oh2 = (col == i2)
    m2 = jnp.sum(jnp.where(oh2, probs2, 0.0), axis=1).reshape(T, 1)
    denom = m1 + m2
    w1 = m1 / denom
    w2 = m2 / denom

    e_a = jnp.concatenate([i1, i2], axis=0)
    w_a = jnp.concatenate([w1, w2], axis=0)

    colA = jax.lax.broadcasted_iota(jnp.int32, (A, E), 1)
    Aoh = (colA == e_a).astype(jnp.float32)

    ri = jax.lax.broadcasted_iota(jnp.int32, (A, A), 0)
    rj = jax.lax.broadcasted_iota(jnp.int32, (A, A), 1)
    L = (rj < ri).astype(jnp.float32)
    pref = jnp.dot(L, Aoh, preferred_element_type=jnp.float32)
    rank = jnp.sum(pref * Aoh, axis=1).reshape(A, 1)

    counts = jnp.sum(Aoh, axis=0).reshape(1, E)
    ntiles = jnp.floor((counts + (R - 1)) * (1.0 / R))
    ui = jax.lax.broadcasted_iota(jnp.int32, (E, E), 0)
    uj = jax.lax.broadcasted_iota(jnp.int32, (E, E), 1)
    U = (ui < uj).astype(jnp.float32)
    first_tile = jnp.dot(ntiles, U, preferred_element_type=jnp.float32)
    cum_incl = first_tile + ntiles

    ft_a = jnp.dot(Aoh, first_tile.reshape(E, 1),
                   preferred_element_type=jnp.float32)
    pos = ft_a * R + rank

    t8 = jax.lax.broadcasted_iota(jnp.int32, (E, NT), 1)
    cmp = (cum_incl.reshape(E, 1).astype(jnp.int32) <= t8).astype(jnp.int32)
    te = jnp.minimum(jnp.sum(cmp, axis=0).reshape(1, NT), E - 1)
    te_ref[...] = te

    prow = jax.lax.broadcasted_iota(jnp.int32, (PR, A), 0)
    pos_i = pos.astype(jnp.int32)
    S = (prow == pos_i.reshape(1, A)).astype(jnp.float32)

    x2 = jnp.concatenate([x, x], axis=0)
    px_ref[...] = jnp.dot(S, x2, preferred_element_type=jnp.float32)

    W2 = S * w_a.reshape(1, A)
    ct_ref[...] = W2[:, :T] + W2[:, T:]


def _moe_kernel(te_ref, px_ref, g_ref, u_ref, d_ref, ct_ref, o_ref):
    t = pl.program_id(0)
    k = pl.program_id(1)

    @pl.when(jnp.logical_and(t == 0, k == 0))
    def _init():
        o_ref[...] = jnp.zeros_like(o_ref)

    xt = px_ref[...]
    g = jnp.dot(xt, g_ref[0], preferred_element_type=jnp.float32)
    u = jnp.dot(xt, u_ref[0], preferred_element_type=jnp.float32)
    h = (g * jax.lax.logistic(g)) * u
    y = jnp.dot(h, d_ref[0], preferred_element_type=jnp.float32)
    ct = ct_ref[...]
    o_ref[...] += jax.lax.dot_general(
        ct, y, (((0,), (0,)), ((), ())),
        preferred_element_type=jnp.float32)


@functools.partial(jax.jit, static_argnames=())
def kernel(hidden_states, router_weight, gate_proj, up_proj, down_proj):
    b, s, d = hidden_states.shape
    x = hidden_states.reshape(T, D)

    px, ct, te = pl.pallas_call(
        _router_kernel,
        out_shape=[
            jax.ShapeDtypeStruct((PR, D), jnp.float32),
            jax.ShapeDtypeStruct((PR, T), jnp.float32),
            jax.ShapeDtypeStruct((1, NT), jnp.int32),
        ],
    )(x, router_weight)

    KS = 2
    FK = F // KS
    out = pl.pallas_call(
        _moe_kernel,
        grid_spec=pltpu.PrefetchScalarGridSpec(
            num_scalar_prefetch=1,
            grid=(NT, KS),
            in_specs=[
                pl.BlockSpec((R, D), lambda t, k, te: (t, 0)),
                pl.BlockSpec((1, D, FK), lambda t, k, te: (te[t], 0, k)),
                pl.BlockSpec((1, D, FK), lambda t, k, te: (te[t], 0, k)),
                pl.BlockSpec((1, FK, D), lambda t, k, te: (te[t], k, 0)),
                pl.BlockSpec((R, T), lambda t, k, te: (t, 0)),
            ],
            out_specs=pl.BlockSpec((T, D), lambda t, k, te: (0, 0)),
        ),
        out_shape=jax.ShapeDtypeStruct((T, D), jnp.float32),
    )(te.reshape(NT), px, gate_proj, up_proj, down_proj, ct)

    return out.reshape(b, s, d)


# speedup vs baseline: 1.4688x; 1.4688x over previous
import functools

import jax
import jax.numpy as jnp
from jax.experimental import pallas as pl
from jax.experimental.pallas import tpu as pltpu

E = 64
K = 2
D = 1024
F = 768
T = 128
A = T * K
R = 8
NT = (A - E) // R + E
PR = NT * R


def _router_kernel(x_ref, rw_ref, px_ref, ct_ref, te_ref, ntot_ref):
    x = x_ref[...]
    logits = jnp.dot(x, rw_ref[...], preferred_element_type=jnp.float32)
    probs = jax.nn.softmax(logits, axis=-1)

    col = jax.lax.broadcasted_iota(jnp.int32, (T, E), 1)
    i1 = jnp.argmax(probs, axis=1).reshape(T, 1)
    oh1 = (col == i1)
    m1 = jnp.sum(jnp.where(oh1, probs, 0.0), axis=1).reshape(T, 1)
    probs2 = jnp.where(oh1, -1.0, probs)
    i2 = jnp.argmax(probs2, axis=1).reshape(T, 1)
    oh2 = (col == i2)
    m2 = jnp.sum(jnp.where(oh2, probs2, 0.0), axis=1).reshape(T, 1)
    denom = m1 + m2
    w1 = m1 / denom
    w2 = m2 / denom

    e_a = jnp.concatenate([i1, i2], axis=0)
    w_a = jnp.concatenate([w1, w2], axis=0)

    colA = jax.lax.broadcasted_iota(jnp.int32, (A, E), 1)
    Aoh = (colA == e_a).astype(jnp.float32)

    ri = jax.lax.broadcasted_iota(jnp.int32, (A, A), 0)
    rj = jax.lax.broadcasted_iota(jnp.int32, (A, A), 1)
    L = (rj < ri).astype(jnp.float32)
    pref = jnp.dot(L, Aoh, preferred_element_type=jnp.float32)
    rank = jnp.sum(pref * Aoh, axis=1).reshape(A, 1)

    counts = jnp.sum(Aoh, axis=0).reshape(1, E)
    ntiles = jnp.floor((counts + (R - 1)) * (1.0 / R))
    ui = jax.lax.broadcasted_iota(jnp.int32, (E, E), 0)
    uj = jax.lax.broadcasted_iota(jnp.int32, (E, E), 1)
    U = (ui < uj).astype(jnp.float32)
    first_tile = jnp.dot(ntiles, U, preferred_element_type=jnp.float32)
    cum_incl = first_tile + ntiles

    ft_a = jnp.dot(Aoh, first_tile.reshape(E, 1),
                   preferred_element_type=jnp.float32)
    pos = ft_a * R + rank

    t8 = jax.lax.broadcasted_iota(jnp.int32, (E, NT), 1)
    cmp = (cum_incl.reshape(E, 1).astype(jnp.int32) <= t8).astype(jnp.int32)
    te = jnp.minimum(jnp.sum(cmp, axis=0).reshape(1, NT), E - 1)
    te_ref[...] = te
    ntot_ref[...] = cum_incl[:, E - 1:E].astype(jnp.int32)

    prow = jax.lax.broadcasted_iota(jnp.int32, (PR, A), 0)
    pos_i = pos.astype(jnp.int32)
    S = (prow == pos_i.reshape(1, A)).astype(jnp.float32)

    x2 = jnp.concatenate([x, x], axis=0)
    px_ref[...] = jnp.dot(S, x2, preferred_element_type=jnp.float32)

    W2 = S * w_a.reshape(1, A)
    ct_ref[...] = W2[:, :T] + W2[:, T:]


def _moe_kernel(te_ref, ntot_ref, px_ref, g_ref, u_ref, d_ref, ct_ref, o_ref):
    t = pl.program_id(0)

    @pl.when(t == 0)
    def _init():
        o_ref[...] = jnp.zeros_like(o_ref)

    @pl.when(t < ntot_ref[0])
    def _compute():
        xt = px_ref[...]
        g = jnp.dot(xt, g_ref[0], preferred_element_type=jnp.float32)
        u = jnp.dot(xt, u_ref[0], preferred_element_type=jnp.float32)
        h = (g * jax.lax.logistic(g)) * u
        y = jnp.dot(h, d_ref[0], preferred_element_type=jnp.float32)
        ct = ct_ref[...]
        o_ref[...] += jax.lax.dot_general(
            ct, y, (((0,), (0,)), ((), ())),
            preferred_element_type=jnp.float32)


@functools.partial(jax.jit, static_argnames=())
def kernel(hidden_states, router_weight, gate_proj, up_proj, down_proj):
    b, s, d = hidden_states.shape
    x = hidden_states.reshape(T, D)

    px, ct, te, ntot = pl.pallas_call(
        _router_kernel,
        out_shape=[
            jax.ShapeDtypeStruct((PR, D), jnp.float32),
            jax.ShapeDtypeStruct((PR, T), jnp.float32),
            jax.ShapeDtypeStruct((1, NT), jnp.int32),
            jax.ShapeDtypeStruct((1, 1), jnp.int32),
        ],
    )(x, router_weight)

    out = pl.pallas_call(
        _moe_kernel,
        grid_spec=pltpu.PrefetchScalarGridSpec(
            num_scalar_prefetch=2,
            grid=(NT,),
            in_specs=[
                pl.BlockSpec((R, D), lambda t, te, nt: (t, 0)),
                pl.BlockSpec((1, D, F), lambda t, te, nt: (te[t], 0, 0)),
                pl.BlockSpec((1, D, F), lambda t, te, nt: (te[t], 0, 0)),
                pl.BlockSpec((1, F, D), lambda t, te, nt: (te[t], 0, 0)),
                pl.BlockSpec((R, T), lambda t, te, nt: (t, 0)),
            ],
            out_specs=pl.BlockSpec((T, D), lambda t, te, nt: (0, 0)),
        ),
        out_shape=jax.ShapeDtypeStruct((T, D), jnp.float32),
    )(te.reshape(NT), ntot.reshape(1), px, gate_proj, up_proj, down_proj, ct)

    return out.reshape(b, s, d)
